# baseline (speedup 1.0000x reference)
import functools

import jax
import jax.numpy as jnp
from jax import lax
from jax.experimental import pallas as pl
from jax.experimental.pallas import tpu as pltpu
from jax.experimental.pallas import tpu_sc as plsc

_NUM_CORES = 2
_NUM_SUBCORES = 16
_NW = _NUM_CORES * _NUM_SUBCORES

_BT = 128
_NBUF = 4


@functools.lru_cache(maxsize=None)
def _build(B, H, D):
  n_units = (B // _BT) * H
  u_per_w = n_units // _NW
  nbt = B // _BT
  dt = D // 8
  mesh = plsc.VectorSubcoreMesh(
      core_axis_name="c", subcore_axis_name="s",
      num_cores=_NUM_CORES, num_subcores=_NUM_SUBCORES)

  @functools.partial(
      pl.kernel,
      out_type=jax.ShapeDtypeStruct((H, dt, nbt, 8, _BT), jnp.float32),
      mesh=mesh,
      scratch_types=[
          pltpu.VMEM((u_per_w, _BT), jnp.int32),
          pltpu.VMEM((_NBUF, _BT, D), jnp.float32),
          pltpu.VMEM((_NBUF, dt, 8, _BT), jnp.float32),
          pltpu.SemaphoreType.DMA((_NBUF,)),
          pltpu.SemaphoreType.DMA((_NBUF,)),
      ],
      compiler_params=pltpu.CompilerParams(
          use_tc_tiling_on_sc=False, needs_layout_passes=False),
  )
  def k(word_hbm, table_hbm, out_hbm, idx_v, rows_v, t_v, gsem, osem):
    wid = lax.axis_index("s") * _NUM_CORES + lax.axis_index("c")
    ubase = wid * u_per_w
    pltpu.sync_copy(word_hbm.at[wid], idx_v)

    iota = lax.iota(jnp.int32, 16)
    bvecs = [iota + (b0 * 16) for b0 in range(_BT // 16)]

    def gather_start(k_, b):
      pltpu.make_async_copy(
          table_hbm.at[idx_v.at[k_]], rows_v.at[b], gsem.at[b]).start()

    def gather_wait(k_, b):
      pltpu.make_async_copy(
          table_hbm.at[idx_v.at[k_]], rows_v.at[b], gsem.at[b]).wait()

    def out_copy(k_, b):
      u = ubase + k_
      h = u // nbt
      bt = u % nbt
      return pltpu.make_async_copy(
          t_v.at[b], out_hbm.at[h, :, bt], osem.at[b])

    def transpose(b):
      for f in range(D):
        fvec = jnp.full((16,), f, jnp.int32)
        for b0 in range(_BT // 16):
          v = plsc.load_gather(rows_v.at[b], [bvecs[b0], fvec])
          t_v[b, f // 8, f % 8, pl.ds(b0 * 16, 16)] = v

    for b in range(_NBUF):
      gather_start(b, b)
    for b in range(_NBUF):
      gather_wait(b, b)
      transpose(b)
      out_copy(b, b).start()
      gather_start(b + _NBUF, b)

    @pl.loop(_NBUF, u_per_w - _NBUF, step=_NBUF)
    def _(g):
      for b in range(_NBUF):
        k_ = g + b
        gather_wait(k_, b)
        out_copy(k_ - _NBUF, b).wait()
        transpose(b)
        out_copy(k_, b).start()
        gather_start(k_ + _NBUF, b)

    for b in range(_NBUF):
      k_ = u_per_w - _NBUF + b
      gather_wait(k_, b)
      out_copy(k_ - _NBUF, b).wait()
      transpose(b)
      out_copy(k_, b).start()
    for b in range(_NBUF):
      out_copy(u_per_w - _NBUF + b, b).wait()

  return k


def kernel(word, table):
  batch, hist = word.shape
  d = table.shape[1]
  n_units = (batch // _BT) * hist
  assert n_units % (_NW * _NBUF) == 0
  w3 = jnp.transpose(word.astype(jnp.int32)).reshape(_NW, n_units // _NW, _BT)
  o = _build(batch, hist, d)(w3, table)
  return jnp.transpose(o, (2, 4, 0, 1, 3)).reshape(batch, hist, d)


# speedup vs baseline: 1.4195x; 1.4195x over previous
import functools

import jax
import jax.numpy as jnp
from jax import lax
from jax.experimental import pallas as pl
from jax.experimental.pallas import tpu as pltpu
from jax.experimental.pallas import tpu_sc as plsc

_NUM_CORES = 2
_NUM_SUBCORES = 16
_NW = _NUM_CORES * _NUM_SUBCORES

_BT = 128
_NBUF = 4


@functools.lru_cache(maxsize=None)
def _build(B, H, D):
  n_units = (B // _BT) * H
  u_per_w = n_units // _NW
  nbt = B // _BT
  dt = D // 8
  mesh = plsc.VectorSubcoreMesh(
      core_axis_name="c", subcore_axis_name="s",
      num_cores=_NUM_CORES, num_subcores=_NUM_SUBCORES)

  @functools.partial(
      pl.kernel,
      out_type=jax.ShapeDtypeStruct((H, dt, nbt, 8, _BT), jnp.float32),
      mesh=mesh,
      scratch_types=[
          pltpu.VMEM((u_per_w, _BT), jnp.int32),
          pltpu.VMEM((_NBUF, _BT, D), jnp.float32),
          pltpu.VMEM((_NBUF, dt, 8, _BT), jnp.float32),
          pltpu.SemaphoreType.DMA((_NBUF,)),
          pltpu.SemaphoreType.DMA((_NBUF,)),
      ],
      compiler_params=pltpu.CompilerParams(
          use_tc_tiling_on_sc=False, needs_layout_passes=False,
          disable_bounds_checks=True),
  )
  def k(word_hbm, table_hbm, out_hbm, idx_v, rows_v, t_v, gsem, osem):
    wid = lax.axis_index("s") * _NUM_CORES + lax.axis_index("c")
    ubase = wid * u_per_w
    pltpu.sync_copy(word_hbm.at[wid], idx_v)

    iota = lax.iota(jnp.int32, 16)
    bvecs = [iota + (b0 * 16) for b0 in range(_BT // 16)]
    fvecs = [jnp.full((16,), f, jnp.int32) for f in range(D)]

    def gather_start(k_, b):
      pltpu.make_async_copy(
          table_hbm.at[idx_v.at[k_]], rows_v.at[b], gsem.at[b]).start()

    def gather_wait(k_, b):
      pltpu.make_async_copy(
          table_hbm.at[idx_v.at[k_]], rows_v.at[b], gsem.at[b]).wait()

    def out_copy(k_, b):
      u = ubase + k_
      h = u // nbt
      bt = u % nbt
      return pltpu.make_async_copy(
          t_v.at[b], out_hbm.at[h, :, bt], osem.at[b])

    def transpose(b):
      for b0 in range(_BT // 16):
        for fh in range(D // 16):
          vs = [
              plsc.load_gather(rows_v.at[b], [bvecs[b0], fvecs[fh * 16 + fi]])
              for fi in range(16)
          ]
          for fi in range(16):
            f = fh * 16 + fi
            t_v[b, f // 8, f % 8, pl.ds(b0 * 16, 16)] = vs[fi]

    for b in range(_NBUF):
      gather_start(b, b)

    @pl.loop(0, u_per_w, step=_NBUF)
    def _(g):
      for b in range(_NBUF):
        k_ = g + b
        gather_wait(k_, b)

        @pl.when(k_ >= _NBUF)
        def _():
          out_copy(k_ - _NBUF, b).wait()

        transpose(b)
        out_copy(k_, b).start()

        @pl.when(k_ + _NBUF < u_per_w)
        def _():
          gather_start(k_ + _NBUF, b)

    for b in range(_NBUF):
      out_copy(u_per_w - _NBUF + b, b).wait()

  return k


def kernel(word, table):
  batch, hist = word.shape
  d = table.shape[1]
  n_units = (batch // _BT) * hist
  assert n_units % (_NW * _NBUF) == 0
  w3 = jnp.transpose(word.astype(jnp.int32)).reshape(_NW, n_units // _NW, _BT)
  o = _build(batch, hist, d)(w3, table)
  return jnp.transpose(o, (2, 4, 0, 1, 3)).reshape(batch, hist, d)


# speedup vs baseline: 1.6262x; 1.1456x over previous
import functools

import jax
import jax.numpy as jnp
import numpy as np
from jax import lax
from jax.experimental import pallas as pl
from jax.experimental.pallas import tpu as pltpu
from jax.experimental.pallas import tpu_sc as plsc

_NUM_CORES = 2
_NUM_SUBCORES = 16
_NW = _NUM_CORES * _NUM_SUBCORES

_BT = 128
_NBUF = 4


@functools.lru_cache(maxsize=None)
def _build(B, H, D):
  n_units = (B // _BT) * H
  u_per_w = n_units // _NW
  nbt = B // _BT
  dt = D // 8
  mesh = plsc.VectorSubcoreMesh(
      core_axis_name="c", subcore_axis_name="s",
      num_cores=_NUM_CORES, num_subcores=_NUM_SUBCORES)

  @functools.partial(
      pl.kernel,
      out_type=jax.ShapeDtypeStruct((H, dt, nbt, 8 * _BT), jnp.float32),
      mesh=mesh,
      scratch_types=[
          pltpu.VMEM((56, 16), jnp.int32),
          pltpu.VMEM((u_per_w, _BT), jnp.int32),
          pltpu.VMEM((_NBUF, _BT, D), jnp.float32),
          pltpu.VMEM((_NBUF, dt, 8 * _BT), jnp.float32),
          pltpu.SemaphoreType.DMA((_NBUF,)),
          pltpu.SemaphoreType.DMA((_NBUF,)),
      ],
      compiler_params=pltpu.CompilerParams(
          use_tc_tiling_on_sc=False, needs_layout_passes=False,
          disable_bounds_checks=True),
  )
  def k(cv_hbm, word_hbm, table_hbm, out_hbm, cv_v, idx_v, rows_v, t_v,
        gsem, osem):
    wid = lax.axis_index("s") * _NUM_CORES + lax.axis_index("c")
    ubase = wid * u_per_w
    pltpu.sync_copy(cv_hbm, cv_v)
    pltpu.sync_copy(word_hbm.at[wid], idx_v)

    bvecs = [cv_v[b0] for b0 in range(8)]
    dvecs = [cv_v[8 + s] for s in range(16)]
    ftvecs = [cv_v[24 + s] for s in range(16)]
    invecs = [cv_v[40 + s] for s in range(16)]

    def gather_start(k_, b):
      pltpu.make_async_copy(
          table_hbm.at[idx_v.at[k_]], rows_v.at[b], gsem.at[b]).start()

    def gather_wait(k_, b):
      pltpu.make_async_copy(
          table_hbm.at[idx_v.at[k_]], rows_v.at[b], gsem.at[b]).wait()

    def out_copy(k_, b):
      u = ubase + k_
      h = u // nbt
      bt = u % nbt
      return pltpu.make_async_copy(
          t_v.at[b], out_hbm.at[h, :, bt], osem.at[b])

    def transpose(b):
      for b0 in range(_BT // 16):
        for fh in range(D // 16):
          f0 = fh * 16
          vs = [
              plsc.load_gather(rows_v.at[b], [bvecs[b0], dvecs[s] + f0])
              for s in range(16)
          ]
          for s in range(16):
            ftv = ftvecs[s] + (f0 // 8)
            inner = invecs[s] + (b0 * 16)
            plsc.store_scatter(t_v.at[b], [ftv, inner], vs[s])

    for b in range(_NBUF):
      gather_start(b, b)

    @pl.loop(0, u_per_w, step=_NBUF)
    def _(g):
      for b in range(_NBUF):
        k_ = g + b
        gather_wait(k_, b)

        @pl.when(k_ >= _NBUF)
        def _():
          out_copy(k_ - _NBUF, b).wait()

        transpose(b)
        out_copy(k_, b).start()

        @pl.when(k_ + _NBUF < u_per_w)
        def _():
          gather_start(k_ + _NBUF, b)

    for b in range(_NBUF):
      out_copy(u_per_w - _NBUF + b, b).wait()

  return k


def kernel(word, table):
  batch, hist = word.shape
  d = table.shape[1]
  n_units = (batch // _BT) * hist
  assert n_units % (_NW * _NBUF) == 0
  w3 = jnp.transpose(word.astype(jnp.int32)).reshape(_NW, n_units // _NW, _BT)
  lanes = np.arange(16)
  diag = [(lanes + s) % 16 for s in range(16)]
  cv = np.stack(
      [lanes + b0 * 16 for b0 in range(8)]
      + diag
      + [d // 8 for d in diag]
      + [(d % 8) * _BT + lanes for d in diag]).astype(np.int32)
  o = _build(batch, hist, d)(jnp.asarray(cv), w3, table)
  o5 = o.reshape(hist, d // 8, batch // _BT, 8, _BT)
  return jnp.transpose(o5, (2, 4, 0, 1, 3)).reshape(batch, hist, d)
